# Initial kernel scaffold; baseline (speedup 1.0000x reference)
import jax
import jax.numpy as jnp
from jax.experimental import pallas as pl


def kernel(x, edge_index, path_lengths, merge_counts, params):
    raise NotImplementedError("write your pallas kernel here")


# speedup vs baseline: 2.0681x; 2.0681x over previous
import functools
import numpy as np
import jax
import jax.numpy as jnp
from jax import lax
from jax.experimental import pallas as pl
from jax.experimental.pallas import tpu as pltpu
from jax.experimental.pallas import tpu_sc as plsc

HID = 256
NL = 3
DECAY = 0.9
EPS_TAU = 1e-08
EPS_LN = 1e-05
LN_DECAY = float(np.log(DECAY))

NC = 2
NS = 16
LANES = 16
HHID = HID // 2


def _mesh():
    return plsc.VectorSubcoreMesh(core_axis_name="c", subcore_axis_name="s")


def _deg_body(dst_hbm, deg_hbm, dst_v, ones_v, deg_sh, sem,
              *, E, NP, C, EPT, RPT):
    c = lax.axis_index("c")
    s = lax.axis_index("s")

    def fill(i, _):
        for k in range(HHID // LANES):
            ones_v[i, pl.ds(k * LANES, LANES)] = jnp.zeros((LANES,),
                                                           jnp.float32)
        return 0
    lax.fori_loop(0, C, fill, 0)
    zbase = s * RPT
    for t in range(RPT // C):
        pltpu.sync_copy(ones_v, deg_sh.at[pl.ds(zbase + t * C, C)])
    if RPT % C:
        pltpu.sync_copy(ones_v.at[pl.ds(0, RPT % C)],
                        deg_sh.at[pl.ds(zbase + (RPT // C) * C, RPT % C)])

    def fill1(i, _):
        for k in range(HHID // LANES):
            ones_v[i, pl.ds(k * LANES, LANES)] = jnp.ones((LANES,),
                                                          jnp.float32)
        return 0
    lax.fori_loop(0, C, fill1, 0)
    plsc.subcore_barrier()

    base0 = s * EPT

    def chunk(i, _):
        base = base0 + i * C
        pltpu.sync_copy(dst_hbm.at[pl.ds(base, C)], dst_v)
        pltpu.sync_copy(ones_v, deg_sh.at[dst_v], add=True)
        return 0
    lax.fori_loop(0, EPT // C, chunk, 0)

    plsc.subcore_barrier()
    pltpu.sync_copy(deg_sh.at[pl.ds(s * RPT, RPT)],
                    deg_hbm.at[pl.ds(c * NP + s * RPT, RPT)])


def _deg(dst, N):
    E = dst.shape[0]
    NP = ((N + NS * 8 - 1) // (NS * 8)) * (NS * 8)
    C = 80
    EPT = E // NS
    RPT = NP // NS
    body = functools.partial(_deg_body, E=E, NP=NP, C=C, EPT=EPT, RPT=RPT)
    f = pl.kernel(
        body,
        out_type=jax.ShapeDtypeStruct((NC * NP, HHID), jnp.float32),
        mesh=_mesh(),
        scratch_types=[
            pltpu.VMEM((C,), jnp.int32),
            pltpu.VMEM((C, HHID), jnp.float32),
            pltpu.VMEM_SHARED((NP, HHID), jnp.float32),
            pltpu.SemaphoreType.DMA,
        ],
    )
    return f(dst), NP


def _wp_body(plen_hbm, mc_hbm, scale_hbm, wp_hbm,
             plen_v, mc_v, wq16_v, scale_v,
             *, E, C, EPT):
    c = lax.axis_index("c")
    s = lax.axis_index("s")

    @pl.when(c == 0)
    def _():
        pltpu.sync_copy(scale_hbm, scale_v)
        base0 = s * EPT

        def chunk(i, _):
            base = base0 + i * C
            pltpu.sync_copy(plen_hbm.at[pl.ds(base, C)], plen_v)
            pltpu.sync_copy(mc_hbm.at[pl.ds(base, C)], mc_v)
            for j in range(C // LANES):
                pv = plen_v[pl.ds(j * LANES, LANES)]
                mv = mc_v[pl.ds(j * LANES, LANES)]
                dist = pv * jnp.exp(mv * LN_DECAY)
                w = jnp.exp(dist * scale_v[...])
                for kk in range(LANES):
                    e = j * LANES + kk
                    wq16_v[e, :] = jnp.broadcast_to(w[kk], (LANES,))
            pltpu.sync_copy(wq16_v, wp_hbm.at[pl.ds(base, C)])
            return 0
        lax.fori_loop(0, EPT // C, chunk, 0)


def _wp(path_lengths, merge_counts, scale16):
    E = path_lengths.shape[0]
    C = 80
    EPT = E // NS
    body = functools.partial(_wp_body, E=E, C=C, EPT=EPT)
    f = pl.kernel(
        body,
        out_type=jax.ShapeDtypeStruct((E, LANES), jnp.float32),
        mesh=_mesh(),
        scratch_types=[
            pltpu.VMEM((C,), jnp.float32),
            pltpu.VMEM((C,), jnp.float32),
            pltpu.VMEM((C, LANES), jnp.float32),
            pltpu.VMEM((LANES,), jnp.float32),
        ],
    )
    return f(path_lengths, merge_counts, scale16)


def _msg_body(h_hbm, src_hbm, dst_hbm, wp_hbm, msg_hbm,
              src_v, dst_v, wp_v, rows_v, msg_sh, sem,
              *, N, NP, C, EPT, RPT):
    c = lax.axis_index("c")
    s = lax.axis_index("s")

    def fill_zero(i, _):
        for k in range(HHID // LANES):
            rows_v[i, pl.ds(k * LANES, LANES)] = jnp.zeros((LANES,),
                                                           jnp.float32)
        return 0
    lax.fori_loop(0, C, fill_zero, 0)
    zbase = s * RPT
    for t in range(RPT // C):
        pltpu.sync_copy(rows_v, msg_sh.at[pl.ds(zbase + t * C, C)])
    if RPT % C:
        pltpu.sync_copy(rows_v.at[pl.ds(0, RPT % C)],
                        msg_sh.at[pl.ds(zbase + (RPT // C) * C, RPT % C)])
    plsc.subcore_barrier()

    base0 = s * EPT
    coff = c * N

    def chunk(i, _):
        base = base0 + i * C
        pltpu.sync_copy(src_hbm.at[pl.ds(base, C)], src_v)
        pltpu.sync_copy(dst_hbm.at[pl.ds(base, C)], dst_v)
        pltpu.sync_copy(wp_hbm.at[pl.ds(base, C)], wp_v)
        for j in range(C // LANES):
            src_v[pl.ds(j * LANES, LANES)] = (
                src_v[pl.ds(j * LANES, LANES)] + coff)
        pltpu.async_copy(h_hbm.at[src_v], rows_v, sem).wait()

        def scale(e, _):
            wrow = wp_v[e, :]
            for k in range(HHID // LANES):
                rows_v[e, pl.ds(k * LANES, LANES)] = (
                    rows_v[e, pl.ds(k * LANES, LANES)] * wrow)
            return 0
        lax.fori_loop(0, C, scale, 0)

        pltpu.sync_copy(rows_v, msg_sh.at[dst_v], add=True)
        return 0
    lax.fori_loop(0, EPT // C, chunk, 0)

    plsc.subcore_barrier()
    pltpu.sync_copy(msg_sh.at[pl.ds(s * RPT, RPT)],
                    msg_hbm.at[pl.ds(c * NP + s * RPT, RPT)])


def _msg(h2flat, src, dst, wp, N, NP):
    E = src.shape[0]
    C = 80
    EPT = E // NS
    RPT = NP // NS
    body = functools.partial(_msg_body, N=N, NP=NP, C=C, EPT=EPT, RPT=RPT)
    f = pl.kernel(
        body,
        out_type=jax.ShapeDtypeStruct((NC * NP, HHID), jnp.float32),
        mesh=_mesh(),
        scratch_types=[
            pltpu.VMEM((C,), jnp.int32),
            pltpu.VMEM((C,), jnp.int32),
            pltpu.VMEM((C, LANES), jnp.float32),
            pltpu.VMEM((C, HHID), jnp.float32),
            pltpu.VMEM_SHARED((NP, HHID), jnp.float32),
            pltpu.SemaphoreType.DMA,
        ],
    )
    return f(h2flat, src, dst, wp)


def _proj_split_body(x_ref, w_ref, b_ref, o_ref):
    y = jnp.dot(x_ref[...], w_ref[...],
                preferred_element_type=jnp.float32) + b_ref[...]
    o_ref[0] = y[:, :HHID]
    o_ref[1] = y[:, HHID:]


def _proj_split(x, W, b, BR=512):
    N, D = x.shape
    G = (N + BR - 1) // BR
    return pl.pallas_call(
        _proj_split_body,
        grid=(G,),
        in_specs=[
            pl.BlockSpec((BR, D), lambda i: (i, 0)),
            pl.BlockSpec((D, HID), lambda i: (0, 0)),
            pl.BlockSpec((1, HID), lambda i: (0, 0)),
        ],
        out_specs=pl.BlockSpec((NC, BR, HHID), lambda i: (0, i, 0)),
        out_shape=jax.ShapeDtypeStruct((NC, N, HHID), jnp.float32),
    )(x, W, b.reshape(1, HID))


def _gru_body(h_ref, m_ref, deg_ref, wz_ref, wr_ref, wh_ref, bz_ref, br_ref,
              bh_ref, g_ref, bt_ref, o_ref):
    h = jnp.concatenate([h_ref[0], h_ref[1]], axis=-1)
    m = jnp.concatenate([m_ref[0], m_ref[1]], axis=-1)
    m = m / jnp.maximum(deg_ref[...][:, :1], 1.0)
    hm = jnp.concatenate([h, m], axis=-1)
    z = jax.nn.sigmoid(jnp.dot(hm, wz_ref[...],
                               preferred_element_type=jnp.float32)
                       + bz_ref[...])
    r = jax.nn.sigmoid(jnp.dot(hm, wr_ref[...],
                               preferred_element_type=jnp.float32)
                       + br_ref[...])
    hr = jnp.concatenate([r * h, m], axis=-1)
    ht = jnp.tanh(jnp.dot(hr, wh_ref[...],
                          preferred_element_type=jnp.float32) + bh_ref[...])
    hn = (1.0 - z) * h + z * ht
    mu = jnp.mean(hn, axis=-1, keepdims=True)
    var = jnp.mean((hn - mu) ** 2, axis=-1, keepdims=True)
    y = (hn - mu) / jnp.sqrt(var + EPS_LN) * g_ref[...] + bt_ref[...]
    o_ref[0] = y[:, :HHID]
    o_ref[1] = y[:, HHID:]


def _gru(h2, msg2, deg, lp, BR=512):
    N = h2.shape[1]
    G = (N + BR - 1) // BR
    spec_w = pl.BlockSpec((2 * HID, HID), lambda i: (0, 0))
    spec_b = pl.BlockSpec((1, HID), lambda i: (0, 0))
    return pl.pallas_call(
        _gru_body,
        grid=(G,),
        in_specs=[
            pl.BlockSpec((NC, BR, HHID), lambda i: (0, i, 0)),
            pl.BlockSpec((NC, BR, HHID), lambda i: (0, i, 0)),
            pl.BlockSpec((BR, HHID), lambda i: (i, 0)),
            spec_w, spec_w, spec_w, spec_b, spec_b, spec_b, spec_b, spec_b,
        ],
        out_specs=pl.BlockSpec((NC, BR, HHID), lambda i: (0, i, 0)),
        out_shape=jax.ShapeDtypeStruct((NC, N, HHID), jnp.float32),
    )(h2, msg2, deg, lp['Wz'], lp['Wr'], lp['Wh'],
      lp['bz'].reshape(1, HID), lp['br'].reshape(1, HID),
      lp['bh'].reshape(1, HID), lp['g'].reshape(1, HID),
      lp['bt'].reshape(1, HID))


def _proj_out_body(h_ref, w_ref, b_ref, o_ref):
    h = jnp.concatenate([h_ref[0], h_ref[1]], axis=-1)
    o_ref[...] = jnp.dot(h, w_ref[...],
                         preferred_element_type=jnp.float32) + b_ref[...]


def _proj_out(h2, W, b, BR=512):
    N = h2.shape[1]
    G = (N + BR - 1) // BR
    return pl.pallas_call(
        _proj_out_body,
        grid=(G,),
        in_specs=[
            pl.BlockSpec((NC, BR, HHID), lambda i: (0, i, 0)),
            pl.BlockSpec((HID, HID), lambda i: (0, 0)),
            pl.BlockSpec((1, HID), lambda i: (0, 0)),
        ],
        out_specs=pl.BlockSpec((BR, HID), lambda i: (i, 0)),
        out_shape=jax.ShapeDtypeStruct((N, HID), jnp.float32),
    )(h2, W, b.reshape(1, HID))


def kernel(x, edge_index, path_lengths, merge_counts, params):
    N, D = x.shape
    E = edge_index.shape[1]
    src = edge_index[0].astype(jnp.int32)
    dst = edge_index[1].astype(jnp.int32)

    deg_flat, NP = _deg(dst, N)
    deg = deg_flat[:NP]
    scale16 = jnp.broadcast_to(-1.0 / (params['tau'] + EPS_TAU),
                               (LANES,)).astype(jnp.float32)
    wp = _wp(path_lengths, merge_counts, scale16)

    h2 = _proj_split(x, params['W_in'], params['b_in'])
    for lp in params['layers']:
        msg_flat = _msg(h2.reshape(NC * N, HHID), src, dst, wp, N, NP)
        msg2 = msg_flat.reshape(NC, NP, HHID)
        h2 = _gru(h2, msg2, deg, lp)
    return _proj_out(h2, params['W_out'], params['b_out'])


# speedup vs baseline: 3.4504x; 1.6684x over previous
import functools
import numpy as np
import jax
import jax.numpy as jnp
from jax import lax
from jax.experimental import pallas as pl
from jax.experimental.pallas import tpu as pltpu
from jax.experimental.pallas import tpu_sc as plsc

HID = 256
NL = 3
DECAY = 0.9
EPS_TAU = 1e-08
EPS_LN = 1e-05
LN_DECAY = float(np.log(DECAY))

NC = 2
NS = 16
LANES = 16
HHID = HID // 2
CEDGE = 80


def _mesh():
    return plsc.VectorSubcoreMesh(core_axis_name="c", subcore_axis_name="s")


def _deg_body(dst_hbm, deg_hbm, dst_v, ones_v, deg_sh, sem,
              *, E, NP, C, EPT, RPT):
    c = lax.axis_index("c")
    s = lax.axis_index("s")

    def fill(i, _):
        for k in range(HHID // LANES):
            ones_v[i, pl.ds(k * LANES, LANES)] = jnp.zeros((LANES,),
                                                           jnp.float32)
        return 0
    lax.fori_loop(0, C, fill, 0)
    zbase = s * RPT
    for t in range(RPT // C):
        pltpu.sync_copy(ones_v, deg_sh.at[pl.ds(zbase + t * C, C)])
    if RPT % C:
        pltpu.sync_copy(ones_v.at[pl.ds(0, RPT % C)],
                        deg_sh.at[pl.ds(zbase + (RPT // C) * C, RPT % C)])

    def fill1(i, _):
        for k in range(HHID // LANES):
            ones_v[i, pl.ds(k * LANES, LANES)] = jnp.ones((LANES,),
                                                          jnp.float32)
        return 0
    lax.fori_loop(0, C, fill1, 0)
    plsc.subcore_barrier()

    base0 = s * EPT

    def chunk(i, _):
        base = base0 + i * C
        pltpu.sync_copy(dst_hbm.at[pl.ds(base, C)], dst_v)
        pltpu.sync_copy(ones_v, deg_sh.at[dst_v], add=True)
        return 0
    lax.fori_loop(0, EPT // C, chunk, 0)

    plsc.subcore_barrier()
    pltpu.sync_copy(deg_sh.at[pl.ds(s * RPT, RPT)],
                    deg_hbm.at[pl.ds(c * NP + s * RPT, RPT)])


def _deg(dst, N):
    E = dst.shape[0]
    NP = ((N + NS * 8 - 1) // (NS * 8)) * (NS * 8)
    C = 80
    EPT = E // NS
    RPT = NP // NS
    body = functools.partial(_deg_body, E=E, NP=NP, C=C, EPT=EPT, RPT=RPT)
    f = pl.kernel(
        body,
        out_type=jax.ShapeDtypeStruct((NC * NP, HHID), jnp.float32),
        mesh=_mesh(),
        scratch_types=[
            pltpu.VMEM((C,), jnp.int32),
            pltpu.VMEM((C, HHID), jnp.float32),
            pltpu.VMEM_SHARED((NP, HHID), jnp.float32),
            pltpu.SemaphoreType.DMA,
        ],
    )
    return f(dst), NP


def _wp_body(plen_hbm, mc_hbm, scale_hbm, wp_hbm,
             plen_v, mc_v, wq16_v, scale_v,
             *, E, C, EPT):
    c = lax.axis_index("c")
    s = lax.axis_index("s")

    @pl.when(c == 0)
    def _():
        pltpu.sync_copy(scale_hbm, scale_v)
        base0 = s * EPT

        def chunk(i, _):
            base = base0 + i * C
            pltpu.sync_copy(plen_hbm.at[pl.ds(base, C)], plen_v)
            pltpu.sync_copy(mc_hbm.at[pl.ds(base, C)], mc_v)
            for j in range(C // LANES):
                pv = plen_v[pl.ds(j * LANES, LANES)]
                mv = mc_v[pl.ds(j * LANES, LANES)]
                dist = pv * jnp.exp(mv * LN_DECAY)
                w = jnp.exp(dist * scale_v[...])
                for kk in range(LANES):
                    e = j * LANES + kk
                    wq16_v[pl.ds(e * LANES, LANES)] = (
                        jnp.broadcast_to(w[kk], (LANES,)))
            pltpu.sync_copy(wq16_v, wp_hbm.at[pl.ds(base * LANES, C * LANES)])
            return 0
        lax.fori_loop(0, EPT // C, chunk, 0)


def _wp(path_lengths, merge_counts, scale16):
    E = path_lengths.shape[0]
    C = 80
    EPT = E // NS
    body = functools.partial(_wp_body, E=E, C=C, EPT=EPT)
    f = pl.kernel(
        body,
        out_type=jax.ShapeDtypeStruct((E * LANES,), jnp.float32),
        mesh=_mesh(),
        scratch_types=[
            pltpu.VMEM((C,), jnp.float32),
            pltpu.VMEM((C,), jnp.float32),
            pltpu.VMEM((C * LANES,), jnp.float32),
            pltpu.VMEM((LANES,), jnp.float32),
        ],
    )
    return f(path_lengths, merge_counts, scale16)


def _msg_body(h_hbm, sd_hbm, wp_hbm, msg_hbm,
              sd0, sd1, wp0, wp1, src0, src1, dst0, dst1, rows0, rows1,
              msg_sh, sidx, sgat,
              *, N, NP, C, EPT, RPT):
    c = lax.axis_index("c")
    s = lax.axis_index("s")
    NCH = EPT // C
    SD = 2 * C
    WPC = C * LANES

    def fill_zero(i, _):
        for k in range(HHID // LANES):
            rows0[i, pl.ds(k * LANES, LANES)] = jnp.zeros((LANES,),
                                                          jnp.float32)
        return 0
    lax.fori_loop(0, C, fill_zero, 0)
    zbase = s * RPT
    for t in range(RPT // C):
        pltpu.sync_copy(rows0, msg_sh.at[pl.ds(zbase + t * C, C)])
    if RPT % C:
        pltpu.sync_copy(rows0.at[pl.ds(0, RPT % C)],
                        msg_sh.at[pl.ds(zbase + (RPT // C) * C, RPT % C)])
    plsc.subcore_barrier()

    g0 = s * NCH
    gend = g0 + NCH
    coff = c * N

    def build(sd_v, src_v, dst_v):
        for j in range(C // LANES):
            src_v[pl.ds(j * LANES, LANES)] = (
                sd_v[pl.ds(j * LANES, LANES)] + coff)
            dst_v[pl.ds(j * LANES, LANES)] = sd_v[pl.ds(C + j * LANES, LANES)]

    def issue_idx(ci, sd_v, wp_v):
        pltpu.async_copy(sd_hbm.at[pl.ds(ci * SD, SD)], sd_v, sidx)
        pltpu.async_copy(wp_hbm.at[pl.ds(ci * WPC, WPC)], wp_v, sidx)

    def wait_idx(ci, sd_v, wp_v):
        pltpu.make_async_copy(sd_hbm.at[pl.ds(ci * SD, SD)], sd_v,
                              sidx).wait()
        pltpu.make_async_copy(wp_hbm.at[pl.ds(ci * WPC, WPC)], wp_v,
                              sidx).wait()

    def scale(rows_v, wp_v):
        def sc(e, _):
            wrow = wp_v[pl.ds(e * LANES, LANES)]
            for k in range(HHID // LANES):
                rows_v[e, pl.ds(k * LANES, LANES)] = (
                    rows_v[e, pl.ds(k * LANES, LANES)] * wrow)
            return 0
        lax.fori_loop(0, C, sc, 0)

    bufs = ((sd0, wp0, src0, dst0, rows0), (sd1, wp1, src1, dst1, rows1))

    pltpu.sync_copy(sd_hbm.at[pl.ds(g0 * SD, SD)], sd0)
    pltpu.sync_copy(wp_hbm.at[pl.ds(g0 * WPC, WPC)], wp0)
    build(sd0, src0, dst0)
    pltpu.async_copy(h_hbm.at[src0], rows0, sgat)
    issue_idx(g0 + 1, sd1, wp1)

    def sub(ci, A, B):
        sdA, wpA, srcA, dstA, rowsA = A
        sdB, wpB, srcB, dstB, rowsB = B
        pltpu.make_async_copy(h_hbm.at[srcA], rowsA, sgat).wait()
        scale(rowsA, wpA)
        wait_idx(ci + 1, sdB, wpB)
        build(sdB, srcB, dstB)
        pltpu.async_copy(h_hbm.at[srcB], rowsB, sgat)

        @pl.when(ci + 2 < gend)
        def _():
            issue_idx(ci + 2, sdA, wpA)
        pltpu.sync_copy(rowsA, msg_sh.at[dstA], add=True)

    def body(i, _):
        ci0 = g0 + i * 2
        sub(ci0, bufs[0], bufs[1])
        sub(ci0 + 1, bufs[1], bufs[0])
        return 0
    lax.fori_loop(0, (NCH - 1) // 2, body, 0)

    lb = (NCH - 1) % 2
    _, wpL, srcL, dstL, rowsL = bufs[lb]
    pltpu.make_async_copy(h_hbm.at[srcL], rowsL, sgat).wait()
    scale(rowsL, wpL)
    pltpu.sync_copy(rowsL, msg_sh.at[dstL], add=True)

    plsc.subcore_barrier()
    pltpu.sync_copy(msg_sh.at[pl.ds(s * RPT, RPT)],
                    msg_hbm.at[pl.ds(c * NP + s * RPT, RPT)])


def _msg(h2flat, sd, wp, N, NP):
    E = sd.shape[0] // 2
    C = CEDGE
    EPT = E // NS
    RPT = NP // NS
    assert (EPT // C) % 2 == 1 and EPT % C == 0
    body = functools.partial(_msg_body, N=N, NP=NP, C=C, EPT=EPT, RPT=RPT)
    f = pl.kernel(
        body,
        out_type=jax.ShapeDtypeStruct((NC * NP, HHID), jnp.float32),
        mesh=_mesh(),
        scratch_types=[
            pltpu.VMEM((2 * C,), jnp.int32),
            pltpu.VMEM((2 * C,), jnp.int32),
            pltpu.VMEM((C * LANES,), jnp.float32),
            pltpu.VMEM((C * LANES,), jnp.float32),
            pltpu.VMEM((C,), jnp.int32),
            pltpu.VMEM((C,), jnp.int32),
            pltpu.VMEM((C,), jnp.int32),
            pltpu.VMEM((C,), jnp.int32),
            pltpu.VMEM((C, HHID), jnp.float32),
            pltpu.VMEM((C, HHID), jnp.float32),
            pltpu.VMEM_SHARED((NP, HHID), jnp.float32),
            pltpu.SemaphoreType.DMA,
            pltpu.SemaphoreType.DMA,
        ],
    )
    return f(h2flat, sd, wp)


def _proj_split_body(x_ref, w_ref, b_ref, o_ref):
    y = jnp.dot(x_ref[...], w_ref[...],
                preferred_element_type=jnp.float32) + b_ref[...]
    o_ref[0] = y[:, :HHID]
    o_ref[1] = y[:, HHID:]


def _proj_split(x, W, b, BR=512):
    N, D = x.shape
    G = (N + BR - 1) // BR
    return pl.pallas_call(
        _proj_split_body,
        grid=(G,),
        in_specs=[
            pl.BlockSpec((BR, D), lambda i: (i, 0)),
            pl.BlockSpec((D, HID), lambda i: (0, 0)),
            pl.BlockSpec((1, HID), lambda i: (0, 0)),
        ],
        out_specs=pl.BlockSpec((NC, BR, HHID), lambda i: (0, i, 0)),
        out_shape=jax.ShapeDtypeStruct((NC, N, HHID), jnp.float32),
    )(x, W, b.reshape(1, HID))


def _gru_body(h_ref, m_ref, deg_ref, wz_ref, wr_ref, wh_ref, bz_ref, br_ref,
              bh_ref, g_ref, bt_ref, o_ref):
    h = jnp.concatenate([h_ref[0], h_ref[1]], axis=-1)
    m = jnp.concatenate([m_ref[0], m_ref[1]], axis=-1)
    m = m / jnp.maximum(deg_ref[...][:, :1], 1.0)
    hm = jnp.concatenate([h, m], axis=-1)
    z = jax.nn.sigmoid(jnp.dot(hm, wz_ref[...],
                               preferred_element_type=jnp.float32)
                       + bz_ref[...])
    r = jax.nn.sigmoid(jnp.dot(hm, wr_ref[...],
                               preferred_element_type=jnp.float32)
                       + br_ref[...])
    hr = jnp.concatenate([r * h, m], axis=-1)
    ht = jnp.tanh(jnp.dot(hr, wh_ref[...],
                          preferred_element_type=jnp.float32) + bh_ref[...])
    hn = (1.0 - z) * h + z * ht
    mu = jnp.mean(hn, axis=-1, keepdims=True)
    var = jnp.mean((hn - mu) ** 2, axis=-1, keepdims=True)
    y = (hn - mu) / jnp.sqrt(var + EPS_LN) * g_ref[...] + bt_ref[...]
    o_ref[0] = y[:, :HHID]
    o_ref[1] = y[:, HHID:]


def _gru(h2, msg2, deg, lp, BR=512):
    N = h2.shape[1]
    G = (N + BR - 1) // BR
    spec_w = pl.BlockSpec((2 * HID, HID), lambda i: (0, 0))
    spec_b = pl.BlockSpec((1, HID), lambda i: (0, 0))
    return pl.pallas_call(
        _gru_body,
        grid=(G,),
        in_specs=[
            pl.BlockSpec((NC, BR, HHID), lambda i: (0, i, 0)),
            pl.BlockSpec((NC, BR, HHID), lambda i: (0, i, 0)),
            pl.BlockSpec((BR, HHID), lambda i: (i, 0)),
            spec_w, spec_w, spec_w, spec_b, spec_b, spec_b, spec_b, spec_b,
        ],
        out_specs=pl.BlockSpec((NC, BR, HHID), lambda i: (0, i, 0)),
        out_shape=jax.ShapeDtypeStruct((NC, N, HHID), jnp.float32),
    )(h2, msg2, deg, lp['Wz'], lp['Wr'], lp['Wh'],
      lp['bz'].reshape(1, HID), lp['br'].reshape(1, HID),
      lp['bh'].reshape(1, HID), lp['g'].reshape(1, HID),
      lp['bt'].reshape(1, HID))


def _proj_out_body(h_ref, w_ref, b_ref, o_ref):
    h = jnp.concatenate([h_ref[0], h_ref[1]], axis=-1)
    o_ref[...] = jnp.dot(h, w_ref[...],
                         preferred_element_type=jnp.float32) + b_ref[...]


def _proj_out(h2, W, b, BR=512):
    N = h2.shape[1]
    G = (N + BR - 1) // BR
    return pl.pallas_call(
        _proj_out_body,
        grid=(G,),
        in_specs=[
            pl.BlockSpec((NC, BR, HHID), lambda i: (0, i, 0)),
            pl.BlockSpec((HID, HID), lambda i: (0, 0)),
            pl.BlockSpec((1, HID), lambda i: (0, 0)),
        ],
        out_specs=pl.BlockSpec((BR, HID), lambda i: (i, 0)),
        out_shape=jax.ShapeDtypeStruct((N, HID), jnp.float32),
    )(h2, W, b.reshape(1, HID))


def kernel(x, edge_index, path_lengths, merge_counts, params):
    N, D = x.shape
    E = edge_index.shape[1]
    src = edge_index[0].astype(jnp.int32)
    dst = edge_index[1].astype(jnp.int32)

    deg_flat, NP = _deg(dst, N)
    deg = deg_flat[:NP]
    scale16 = jnp.broadcast_to(-1.0 / (params['tau'] + EPS_TAU),
                               (LANES,)).astype(jnp.float32)
    wp = _wp(path_lengths, merge_counts, scale16)
    sd = jnp.concatenate([src.reshape(-1, CEDGE), dst.reshape(-1, CEDGE)],
                         axis=1).reshape(-1)

    h2 = _proj_split(x, params['W_in'], params['b_in'])
    for lp in params['layers']:
        msg_flat = _msg(h2.reshape(NC * N, HHID), sd, wp, N, NP)
        msg2 = msg_flat.reshape(NC, NP, HHID)
        h2 = _gru(h2, msg2, deg, lp)
    return _proj_out(h2, params['W_out'], params['b_out'])
